# baseline (speedup 1.0000x reference)
import functools

import jax
import jax.numpy as jnp
from jax import lax
from jax.experimental import pallas as pl
from jax.experimental.pallas import tpu as pltpu
from jax.experimental.pallas import tpu_sc as plsc

_EMBED = 64
_HIDDEN = 128
_NC, _NS = 2, 16
_NW = _NC * _NS


def _sc_gather2(user_table, user_idx, movie_table, movie_idx):
    b = user_idx.shape[0]
    b_per_w = b // _NW
    mesh = plsc.VectorSubcoreMesh(core_axis_name="c", subcore_axis_name="s")

    @functools.partial(
        pl.kernel,
        out_type=[
            jax.ShapeDtypeStruct((b, _EMBED), jnp.float32),
            jax.ShapeDtypeStruct((b, _EMBED), jnp.float32),
        ],
        mesh=mesh,
        scratch_types=[
            pltpu.VMEM((b_per_w,), jnp.int32),
            pltpu.VMEM((b_per_w, _EMBED), jnp.float32),
            pltpu.VMEM((b_per_w,), jnp.int32),
            pltpu.VMEM((b_per_w, _EMBED), jnp.float32),
            pltpu.SemaphoreType.DMA,
            pltpu.SemaphoreType.DMA,
        ],
        compiler_params=pltpu.CompilerParams(use_tc_tiling_on_sc=False),
    )
    def k(ut_hbm, ui_hbm, mt_hbm, mi_hbm, uo_hbm, mo_hbm,
          ui_v, ur_v, mi_v, mr_v, usem, msem):
        wid = lax.axis_index("s") * _NC + lax.axis_index("c")
        base = wid * b_per_w
        pltpu.sync_copy(ui_hbm.at[pl.ds(base, b_per_w)], ui_v)
        pltpu.sync_copy(mi_hbm.at[pl.ds(base, b_per_w)], mi_v)
        cu = pltpu.async_copy(ut_hbm.at[ui_v], ur_v, usem)
        cm = pltpu.async_copy(mt_hbm.at[mi_v], mr_v, msem)
        cu.wait()
        pltpu.sync_copy(ur_v, uo_hbm.at[pl.ds(base, b_per_w)])
        cm.wait()
        pltpu.sync_copy(mr_v, mo_hbm.at[pl.ds(base, b_per_w)])

    return k(user_table, user_idx, movie_table, movie_idx)


def _mlp_body(u_ref, m_ref, w1u_ref, w1m_ref, b1_ref, w2_ref, b2_ref, o_ref):
    h = (
        jnp.dot(u_ref[...], w1u_ref[...], preferred_element_type=jnp.float32)
        + jnp.dot(m_ref[...], w1m_ref[...], preferred_element_type=jnp.float32)
        + b1_ref[...]
    )
    h = jnp.maximum(h, 0.0)
    o_ref[...] = (
        jnp.dot(h, w2_ref[...], preferred_element_type=jnp.float32)
        + b2_ref[0, 0]
    )


def _tc_mlp(u_emb, m_emb, W1, b1, W2, b2):
    b = u_emb.shape[0]
    blk = 2048
    w1u_t = W1[:, :_EMBED].T
    w1m_t = W1[:, _EMBED:].T
    out = pl.pallas_call(
        _mlp_body,
        grid=(b // blk,),
        in_specs=[
            pl.BlockSpec((blk, _EMBED), lambda i: (i, 0)),
            pl.BlockSpec((blk, _EMBED), lambda i: (i, 0)),
            pl.BlockSpec((_EMBED, _HIDDEN), lambda i: (0, 0)),
            pl.BlockSpec((_EMBED, _HIDDEN), lambda i: (0, 0)),
            pl.BlockSpec((1, _HIDDEN), lambda i: (0, 0)),
            pl.BlockSpec((_HIDDEN, 1), lambda i: (0, 0)),
            pl.BlockSpec((1, 1), lambda i: (0, 0)),
        ],
        out_specs=pl.BlockSpec((blk, 1), lambda i: (i, 0)),
        out_shape=jax.ShapeDtypeStruct((b, 1), jnp.float32),
    )(u_emb, m_emb, w1u_t, w1m_t, b1.reshape(1, _HIDDEN),
      W2.reshape(_HIDDEN, 1), b2.reshape(1, 1))
    return out.reshape(b)


def kernel(user_input, movie_input, user_table, movie_table, W1, b1, W2, b2):
    u_emb, m_emb = _sc_gather2(
        user_table, user_input.astype(jnp.int32),
        movie_table, movie_input.astype(jnp.int32))
    return _tc_mlp(u_emb, m_emb, W1, b1, W2, b2)
